# Initial kernel scaffold; baseline (speedup 1.0000x reference)
import functools

import jax
import jax.numpy as jnp
from jax import lax
from jax.experimental import pallas as pl
from jax.experimental.pallas import tpu as pltpu
from jax.experimental.pallas import tpu_sc as plsc

B = 4096
L = 200
V = 1000
D = 64

NC, NS, LANES = 2, 16, 16
NW = NC * NS
T = B * L
TW = T // NW
RW = B // NW
G = 128
NG = TW // G
NBUF = 4
ZROW = V


def _sc_body(table_hbm, tok_hbm, ts_hbm, lut_hbm, out_hbm,
             tok_v, lut_v, ts_v, idx_v, buf_v, gsem, *psems):
    wid = lax.axis_index("s") * NC + lax.axis_index("c")
    tok_base = wid * TW
    row_base = wid * RW

    pltpu.sync_copy(tok_hbm.at[pl.ds(tok_base, TW)], tok_v)
    pltpu.sync_copy(lut_hbm, lut_v)
    pltpu.sync_copy(ts_hbm.at[pl.ds(row_base, RW)], ts_v)

    def step(og, carry):
        for b in range(NBUF):
            g = og * NBUF + b
            @pl.when(og > 0)
            def _wait():
                pltpu.make_async_copy(
                    buf_v.at[b], out_hbm.at[pl.ds(0, G)], psems[b]).wait()
            irow = idx_v.at[b]
            base = g * G
            for i in range(G // LANES):
                o = base + i * LANES
                tok = tok_v[pl.ds(o, LANES)]
                ids = plsc.load_gather(lut_v, [tok])
                j = o + lax.iota(jnp.int32, LANES)
                col = jax.lax.rem(j, L)
                row = jax.lax.div(j, L)
                tsr = plsc.load_gather(ts_v, [row])
                masked = col < tsr
                irow[pl.ds(i * LANES, LANES)] = jnp.where(masked, ids, ZROW)
            pltpu.async_copy(table_hbm.at[irow], buf_v.at[b], gsem).wait()
            pltpu.async_copy(
                buf_v.at[b], out_hbm.at[pl.ds(tok_base + base, G)], psems[b])
        return carry

    lax.fori_loop(0, NG // NBUF, step, 0)
    for b in range(NBUF):
        pltpu.make_async_copy(
            buf_v.at[b], out_hbm.at[pl.ds(0, G)], psems[b]).wait()


_sc_gather = functools.partial(
    pl.kernel,
    out_type=jax.ShapeDtypeStruct((T, D), jnp.float32),
    mesh=plsc.VectorSubcoreMesh(
        core_axis_name="c", subcore_axis_name="s",
        num_cores=NC, num_subcores=NS),
    scratch_types=[
        pltpu.VMEM((TW,), jnp.int32),
        pltpu.VMEM((V,), jnp.int32),
        pltpu.VMEM((RW,), jnp.int32),
        pltpu.VMEM((NBUF, G), jnp.int32),
        pltpu.VMEM((NBUF, G, D), jnp.float32),
    ] + [pltpu.SemaphoreType.DMA] * (1 + NBUF),
)(_sc_body)


def _tc_body(rl_ref, msl_ref, ts_ref, mask_ref):
    ts = jnp.minimum(jnp.minimum(rl_ref[...], msl_ref[...]), L).astype(jnp.int32)
    ts_ref[...] = ts
    col = lax.broadcasted_iota(jnp.int32, (B, L), 1)
    mask_ref[...] = col < ts


_tc_mask = pl.pallas_call(
    _tc_body,
    out_shape=(
        jax.ShapeDtypeStruct((B, 1), jnp.int32),
        jax.ShapeDtypeStruct((B, L), jnp.bool_),
    ),
)


def kernel(tokens, row_lengths, max_sequence_length, lookup_table, embeddings):
    msl = jnp.asarray(max_sequence_length, jnp.int32).reshape(1, 1)
    ts2d, mask = _tc_mask(row_lengths.reshape(B, 1).astype(jnp.int32), msl)
    time_steps = ts2d.reshape(B)
    table_ext = jnp.concatenate(
        [embeddings, jnp.zeros((8, D), jnp.float32)], axis=0)
    out_flat = _sc_gather(table_ext, tokens.reshape(T),
                          time_steps, lookup_table)
    return out_flat.reshape(B, L, D), mask, time_steps


# speedup vs baseline: 1.1067x; 1.1067x over previous
import functools

import jax
import jax.numpy as jnp
from jax import lax
from jax.experimental import pallas as pl
from jax.experimental.pallas import tpu as pltpu
from jax.experimental.pallas import tpu_sc as plsc

B = 4096
L = 200
V = 1000
D = 64

NC, NS, LANES = 2, 16, 16
NW = NC * NS
T = B * L
TW = T // NW
RW = B // NW
G = 128
NG = TW // G
NBUF = 4
ZROW = V


def _sc_body(table_hbm, tok_hbm, ts_hbm, lut_hbm, out_hbm,
             tok_v, lut_v, ts_v, idx_v, buf_v, gsem, p0, p1, p2, p3):
    psems = (p0, p1, p2, p3)
    wid = lax.axis_index("s") * NC + lax.axis_index("c")
    tok_base = wid * TW
    row_base = wid * RW

    pltpu.sync_copy(tok_hbm.at[pl.ds(tok_base, TW)], tok_v)
    pltpu.sync_copy(lut_hbm, lut_v)
    pltpu.sync_copy(ts_hbm.at[pl.ds(row_base, RW)], ts_v)

    def step(og, carry):
        for b in range(NBUF):
            g = og * NBUF + b
            @pl.when(og > 0)
            def _wait():
                pltpu.make_async_copy(
                    buf_v.at[b], out_hbm.at[pl.ds(0, G)], psems[b]).wait()
            irow = idx_v.at[b]
            base = g * G
            for i in range(G // LANES):
                o = base + i * LANES
                tok = tok_v[pl.ds(o, LANES)]
                ids = plsc.load_gather(lut_v, [tok])
                j = o + lax.iota(jnp.int32, LANES)
                col = jax.lax.rem(j, L)
                row = jax.lax.div(j, L)
                tsr = plsc.load_gather(ts_v, [row])
                masked = col < tsr
                irow[pl.ds(i * LANES, LANES)] = jnp.where(masked, ids, ZROW)
            pltpu.async_copy(table_hbm.at[irow], buf_v.at[b], gsem).wait()
            pltpu.async_copy(
                buf_v.at[b], out_hbm.at[pl.ds(tok_base + base, G)], psems[b])
        return carry

    lax.fori_loop(0, NG // NBUF, step, 0)
    for b in range(NBUF):
        pltpu.make_async_copy(
            buf_v.at[b], out_hbm.at[pl.ds(0, G)], psems[b]).wait()


_sc_gather = functools.partial(
    pl.kernel,
    out_type=jax.ShapeDtypeStruct((T, D), jnp.float32),
    mesh=plsc.VectorSubcoreMesh(
        core_axis_name="c", subcore_axis_name="s",
        num_cores=NC, num_subcores=NS),
    scratch_types=[
        pltpu.VMEM((TW,), jnp.int32),
        pltpu.VMEM((V,), jnp.int32),
        pltpu.VMEM((RW,), jnp.int32),
        pltpu.VMEM((NBUF, G), jnp.int32),
        pltpu.VMEM((NBUF, G, D), jnp.float32),
    ] + [pltpu.SemaphoreType.DMA] * (1 + NBUF),
    compiler_params=pltpu.CompilerParams(
        needs_layout_passes=False, use_tc_tiling_on_sc=False),
)(_sc_body)


def _tc_body(rl_ref, msl_ref, ts_ref, mask_ref):
    ts = jnp.minimum(jnp.minimum(rl_ref[...], msl_ref[...]), L).astype(jnp.int32)
    ts_ref[...] = ts
    col = lax.broadcasted_iota(jnp.int32, (B, L), 1)
    mask_ref[...] = col < ts


_tc_mask = pl.pallas_call(
    _tc_body,
    out_shape=(
        jax.ShapeDtypeStruct((B, 1), jnp.int32),
        jax.ShapeDtypeStruct((B, L), jnp.bool_),
    ),
)


def kernel(tokens, row_lengths, max_sequence_length, lookup_table, embeddings):
    msl = jnp.asarray(max_sequence_length, jnp.int32).reshape(1, 1)
    ts2d, mask = _tc_mask(row_lengths.reshape(B, 1).astype(jnp.int32), msl)
    time_steps = ts2d.reshape(B)
    table_ext = jnp.concatenate(
        [embeddings, jnp.zeros((8, D), jnp.float32)], axis=0)
    out_flat = _sc_gather(table_ext, tokens.reshape(T),
                          time_steps, lookup_table)
    return out_flat.reshape(B, L, D), mask, time_steps


# speedup vs baseline: 10.2589x; 9.2695x over previous
import functools

import jax
import jax.numpy as jnp
from jax import lax
from jax.experimental import pallas as pl
from jax.experimental.pallas import tpu as pltpu
from jax.experimental.pallas import tpu_sc as plsc

B = 4096
L = 200
V = 1000
D = 64

NC, NS, LANES = 2, 16, 16
NW = NC * NS
T = B * L
TW = T // NW
RW = B // NW
CH = 400
NCH = TW // CH
NBUF = 2
VP = V + 8
ZROW = V


def _sc_body(table_hbm, tok_hbm, ts_hbm, lut_hbm, out_hbm,
             table_v, lut_v, ts_v, tok_v, stage_v,
             t0, t1, p0, p1):
    tsems = (t0, t1)
    psems = (p0, p1)
    wid = lax.axis_index("s") * NC + lax.axis_index("c")
    tok_base = wid * TW
    row_base = wid * RW

    pltpu.sync_copy(table_hbm, table_v)
    pltpu.sync_copy(lut_hbm, lut_v)
    pltpu.sync_copy(ts_hbm.at[pl.ds(row_base, RW)], ts_v)
    for b in range(NBUF):
        pltpu.async_copy(tok_hbm.at[pl.ds(tok_base + b * CH, CH)],
                         tok_v.at[b], tsems[b])

    iota = lax.iota(jnp.int32, LANES)

    def step(och, carry):
        for b in range(NBUF):
            ch = och * NBUF + b
            tvec = tok_v.at[b]
            svec = stage_v.at[b]
            pltpu.make_async_copy(
                tok_hbm.at[pl.ds(0, CH)], tvec, tsems[b]).wait()
            @pl.when(och > 0)
            def _drain():
                pltpu.make_async_copy(
                    svec, out_hbm.at[pl.ds(0, CH * D)], psems[b]).wait()

            def inner(i, car):
                tok = tvec[pl.ds(i * LANES, LANES)]
                ids = plsc.load_gather(lut_v, [tok])
                j = ch * CH + i * LANES + iota
                col = jax.lax.rem(j, L)
                row = jax.lax.div(j, L)
                tsr = plsc.load_gather(ts_v, [row])
                fid = jnp.where(col < tsr, ids, ZROW) * D
                tb = (i * LANES + iota) * D
                for c in range(D):
                    cv = (iota + c) & (D - 1)
                    val = plsc.load_gather(table_v, [fid + cv])
                    plsc.store_scatter(svec, [tb + cv], val)
                return car

            lax.fori_loop(0, CH // LANES, inner, 0)
            @pl.when(ch + NBUF < NCH)
            def _pref():
                pltpu.async_copy(
                    tok_hbm.at[pl.ds(tok_base + (ch + NBUF) * CH, CH)],
                    tvec, tsems[b])
            pltpu.async_copy(
                svec, out_hbm.at[pl.ds((tok_base + ch * CH) * D, CH * D)],
                psems[b])
        return carry

    lax.fori_loop(0, NCH // NBUF, step, 0)
    for b in range(NBUF):
        pltpu.make_async_copy(
            stage_v.at[b], out_hbm.at[pl.ds(0, CH * D)], psems[b]).wait()


_sc_gather = functools.partial(
    pl.kernel,
    out_type=jax.ShapeDtypeStruct((T * D,), jnp.float32),
    mesh=plsc.VectorSubcoreMesh(
        core_axis_name="c", subcore_axis_name="s",
        num_cores=NC, num_subcores=NS),
    scratch_types=[
        pltpu.VMEM((VP * D,), jnp.float32),
        pltpu.VMEM((V,), jnp.int32),
        pltpu.VMEM((RW,), jnp.int32),
        pltpu.VMEM((NBUF, CH), jnp.int32),
        pltpu.VMEM((NBUF, CH * D), jnp.float32),
    ] + [pltpu.SemaphoreType.DMA] * (2 * NBUF),
    compiler_params=pltpu.CompilerParams(
        needs_layout_passes=False, use_tc_tiling_on_sc=False),
)(_sc_body)


def _tc_body(rl_ref, msl_ref, ts_ref, mask_ref):
    ts = jnp.minimum(jnp.minimum(rl_ref[...], msl_ref[...]), L).astype(jnp.int32)
    ts_ref[...] = ts
    col = lax.broadcasted_iota(jnp.int32, (B, L), 1)
    mask_ref[...] = col < ts


_tc_mask = pl.pallas_call(
    _tc_body,
    out_shape=(
        jax.ShapeDtypeStruct((B, 1), jnp.int32),
        jax.ShapeDtypeStruct((B, L), jnp.bool_),
    ),
)


def kernel(tokens, row_lengths, max_sequence_length, lookup_table, embeddings):
    msl = jnp.asarray(max_sequence_length, jnp.int32).reshape(1, 1)
    ts2d, mask = _tc_mask(row_lengths.reshape(B, 1).astype(jnp.int32), msl)
    time_steps = ts2d.reshape(B)
    table_ext = jnp.concatenate(
        [embeddings, jnp.zeros((VP - V, D), jnp.float32)], axis=0)
    out_flat = _sc_gather(table_ext.reshape(VP * D), tokens.reshape(T),
                          time_steps, lookup_table)
    return out_flat.reshape(B, L, D), mask, time_steps
